# Initial kernel scaffold; baseline (speedup 1.0000x reference)
import jax
import jax.numpy as jnp
from jax.experimental import pallas as pl


def kernel(node_feature, batch_node_ids, batch_macro_node_ids):
    raise NotImplementedError("write your pallas kernel here")


# speedup vs baseline: 4.1245x; 4.1245x over previous
import jax
import jax.numpy as jnp
from jax import lax
from jax.experimental import pallas as pl
from jax.experimental.pallas import tpu as pltpu
from jax.experimental.pallas import tpu_sc as plsc

_N_NODES = 10000
_D = 128
_DH = _D // 2
_M = 320000
_S = 5000
_NC, _NS = 2, 16
_S_PAD = 5120
_ROWS_PER_TILE = _S_PAD // _NS
_PER_T = _M // _NS
_C = 80
_NCHUNK = _PER_T // _C
_CW = 16


def _sc_body(table, node_ids, seg_ids, zrow, zcnt, ones,
             psums, pcnts,
             idx_n, idx_s, rows, onebuf, zb, zc, sums_sp, counts_sp, sem):
    cid = lax.axis_index("c")
    sid = lax.axis_index("s")
    base = sid * _PER_T
    r0 = sid * _ROWS_PER_TILE

    pltpu.sync_copy(zrow, zb)
    pltpu.sync_copy(zcnt, zc)
    pltpu.sync_copy(zb, sums_sp.at[pl.ds(r0, _ROWS_PER_TILE)])
    pltpu.sync_copy(zc, counts_sp.at[pl.ds(r0, _ROWS_PER_TILE)])
    pltpu.sync_copy(ones, onebuf)
    plsc.subcore_barrier()

    def step(i, carry):
        off = base + i * _C
        pltpu.sync_copy(node_ids.at[pl.ds(off, _C)], idx_n)
        pltpu.sync_copy(seg_ids.at[pl.ds(off, _C)], idx_s)
        for j in range(_C // 16):
            v = idx_n[pl.ds(16 * j, 16)]
            idx_n[pl.ds(16 * j, 16)] = v + v + cid
        pltpu.async_copy(table.at[idx_n], rows, sem).wait()
        pltpu.sync_copy(rows, sums_sp.at[idx_s], add=True)
        @pl.when((i < _NCHUNK // 2) == (cid == 0))
        def _():
            pltpu.sync_copy(onebuf, counts_sp.at[idx_s], add=True)
        return carry

    lax.fori_loop(0, _NCHUNK, step, 0)
    plsc.subcore_barrier()

    pltpu.sync_copy(sums_sp.at[pl.ds(r0, _ROWS_PER_TILE)], zb)
    pltpu.sync_copy(zb, psums.at[cid, pl.ds(r0, _ROWS_PER_TILE)])
    pltpu.sync_copy(counts_sp.at[pl.ds(r0, _ROWS_PER_TILE)], zc)
    pltpu.sync_copy(zc, pcnts.at[cid, pl.ds(r0, _ROWS_PER_TILE)])


def _finalize_body(ps_ref, pc_ref, out_ref):
    c = pc_ref[0, :_S, 0:1] + pc_ref[1, :_S, 0:1]
    inv = 1.0 / jnp.maximum(c, 1.0)
    left = ps_ref[0, :_S, :] * inv
    right = ps_ref[1, :_S, :] * inv
    out_ref[...] = jnp.concatenate([left, right], axis=1)


@jax.jit
def _impl(node_feature, batch_node_ids, batch_macro_node_ids):
    table = node_feature.reshape(_N_NODES * _NC, _DH)
    zrow = jnp.zeros((_ROWS_PER_TILE, _DH), jnp.float32)
    zcnt = jnp.zeros((_ROWS_PER_TILE, _CW), jnp.float32)
    ones = jnp.ones((_C, _CW), jnp.float32)

    mesh = plsc.VectorSubcoreMesh(core_axis_name="c", subcore_axis_name="s")
    psums, pcnts = pl.kernel(
        _sc_body,
        out_type=(
            jax.ShapeDtypeStruct((_NC, _S_PAD, _DH), jnp.float32),
            jax.ShapeDtypeStruct((_NC, _S_PAD, _CW), jnp.float32),
        ),
        mesh=mesh,
        compiler_params=pltpu.CompilerParams(use_tc_tiling_on_sc=False),
        scratch_types=[
            pltpu.VMEM((_C,), jnp.int32),
            pltpu.VMEM((_C,), jnp.int32),
            pltpu.VMEM((_C, _DH), jnp.float32),
            pltpu.VMEM((_C, _CW), jnp.float32),
            pltpu.VMEM((_ROWS_PER_TILE, _DH), jnp.float32),
            pltpu.VMEM((_ROWS_PER_TILE, _CW), jnp.float32),
            pltpu.VMEM_SHARED((_S_PAD, _DH), jnp.float32),
            pltpu.VMEM_SHARED((_S_PAD, _CW), jnp.float32),
            pltpu.SemaphoreType.DMA,
        ],
    )(table, batch_node_ids, batch_macro_node_ids, zrow, zcnt, ones)

    out = pl.pallas_call(
        _finalize_body,
        out_shape=jax.ShapeDtypeStruct((_S, _D), jnp.float32),
    )(psums, pcnts)
    return out


def kernel(node_feature, batch_node_ids, batch_macro_node_ids):
    return _impl(node_feature, batch_node_ids, batch_macro_node_ids)


# speedup vs baseline: 12.6470x; 3.0663x over previous
import jax
import jax.numpy as jnp
from jax import lax
from jax.experimental import pallas as pl
from jax.experimental.pallas import tpu as pltpu
from jax.experimental.pallas import tpu_sc as plsc

_N_NODES = 10000
_D = 128
_DH = _D // 2
_M = 320000
_S = 5000
_NC, _NS = 2, 16
_S_PAD = 5120
_ROWS_PER_TILE = _S_PAD // _NS
_PER_T = _M // _NS
_C = 80
_NCHUNK = _PER_T // _C
_K = 5
_NGRP = _NCHUNK // _K
_CW = 16


def _sc_body(tbl_l, tbl_r, node_ids, seg_ids3, zrow, zcnt, ones,
             psums, pcnts, *scratch):
    idx_n, idx_s, onebuf, cbuf = scratch[:4]
    rows = scratch[4:4 + 2 * _K]
    sums_sp, counts_sp = scratch[4 + 2 * _K:6 + 2 * _K]
    gsem = scratch[6 + 2 * _K:8 + 2 * _K]
    ssem = scratch[8 + 2 * _K:10 + 2 * _K]
    csem = scratch[10 + 2 * _K]

    cid = lax.axis_index("c")
    sid = lax.axis_index("s")
    base = sid * _PER_T
    r0 = sid * _ROWS_PER_TILE

    def sums_sp_slice(j, r0):
        return sums_sp.at[pl.ds(r0 + j * _C, _C)]

    def counts_sp_slice(j, r0):
        return counts_sp.at[pl.ds(r0 + j * _C, _C)]

    def psum_scatter_dst(i):
        return sums_sp.at[idx_s.at[i]]

    def pcnt_scatter_dst(i):
        return counts_sp.at[idx_s.at[i]]

    pltpu.sync_copy(node_ids.at[pl.ds(base, _PER_T)], idx_n)
    pltpu.sync_copy(seg_ids3.at[sid], idx_s)

    pltpu.sync_copy(zrow, rows[0])
    pltpu.sync_copy(zcnt, cbuf)
    for j in range(_ROWS_PER_TILE // _C):
        pltpu.sync_copy(rows[0], sums_sp_slice(j, r0))
        pltpu.sync_copy(cbuf, counts_sp_slice(j, r0))
    pltpu.sync_copy(ones, onebuf)
    plsc.subcore_barrier()

    def issue_gather(i, buf, sem):
        @pl.when(cid == 0)
        def _():
            pltpu.async_copy(tbl_l.at[idx_n.at[pl.ds(i * _C, _C)]], buf, sem)

        @pl.when(cid == 1)
        def _():
            pltpu.async_copy(tbl_r.at[idx_n.at[pl.ds(i * _C, _C)]], buf, sem)

    for j in range(_K):
        issue_gather(j, rows[j], gsem[0])

    def super_body(u, carry):
        for p in (0, 1):
            t = 2 * u + p
            bb = p * _K
            nbb = (1 - p) * _K
            for j in range(_K):
                pltpu.make_async_copy(zrow, rows[bb + j], gsem[p]).wait()
            for j in range(_K):
                i = t * _K + j
                pltpu.async_copy(rows[bb + j], psum_scatter_dst(i), ssem[p],
                                 add=True)

                @pl.when((i < _NCHUNK // 2) == (cid == 0))
                def _():
                    pltpu.async_copy(onebuf, pcnt_scatter_dst(i), csem,
                                     add=True)
            def drain_prev():
                for j in range(_K):
                    pltpu.make_async_copy(zrow, rows[nbb + j],
                                          ssem[1 - p]).wait()

            def issue_next():
                for j in range(_K):
                    issue_gather((t + 1) * _K + j, rows[nbb + j],
                                 gsem[1 - p])

            if p == 1:
                drain_prev()
                pl.when(u < (_NGRP // 2) - 1)(issue_next)
            else:
                pl.when(u >= 1)(drain_prev)
                issue_next()
        return carry

    lax.fori_loop(0, _NGRP // 2, super_body, 0)

    for j in range(_K):
        pltpu.make_async_copy(zrow, rows[_K + j], ssem[1]).wait()

    def drain_counts(i, carry):
        pltpu.make_async_copy(zcnt, cbuf, csem).wait()
        return carry

    lax.fori_loop(0, _NCHUNK // 2, drain_counts, 0)
    plsc.subcore_barrier()

    for j in range(_ROWS_PER_TILE // _C):
        pltpu.sync_copy(sums_sp_slice(j, r0), rows[j])
        pltpu.sync_copy(rows[j], psums.at[cid, pl.ds(r0 + j * _C, _C)])
        pltpu.sync_copy(counts_sp_slice(j, r0), cbuf)
        pltpu.sync_copy(cbuf, pcnts.at[cid, pl.ds(r0 + j * _C, _C)])


def _finalize_body(ps_ref, pc_ref, out_ref):
    c = pc_ref[0, :_S, 0:1] + pc_ref[1, :_S, 0:1]
    inv = 1.0 / jnp.maximum(c, 1.0)
    left = ps_ref[0, :_S, :] * inv
    right = ps_ref[1, :_S, :] * inv
    out_ref[...] = jnp.concatenate([left, right], axis=1)


@jax.jit
def _impl(node_feature, batch_node_ids, batch_macro_node_ids):
    tbl_l = node_feature[:, :_DH]
    tbl_r = node_feature[:, _DH:]
    seg_ids3 = batch_macro_node_ids.reshape(_NS, _NCHUNK, _C)
    zrow = jnp.zeros((_C, _DH), jnp.float32)
    zcnt = jnp.zeros((_C, _CW), jnp.float32)
    ones = jnp.ones((_C, _CW), jnp.float32)

    mesh = plsc.VectorSubcoreMesh(core_axis_name="c", subcore_axis_name="s")
    psums, pcnts = pl.kernel(
        _sc_body,
        out_type=(
            jax.ShapeDtypeStruct((_NC, _S_PAD, _DH), jnp.float32),
            jax.ShapeDtypeStruct((_NC, _S_PAD, _CW), jnp.float32),
        ),
        mesh=mesh,
        compiler_params=pltpu.CompilerParams(use_tc_tiling_on_sc=False),
        scratch_types=[
            pltpu.VMEM((_PER_T,), jnp.int32),
            pltpu.VMEM((_NCHUNK, _C), jnp.int32),
            pltpu.VMEM((_C, _CW), jnp.float32),
            pltpu.VMEM((_C, _CW), jnp.float32),
        ] + [pltpu.VMEM((_C, _DH), jnp.float32) for _ in range(2 * _K)] + [
            pltpu.VMEM_SHARED((_S_PAD, _DH), jnp.float32),
            pltpu.VMEM_SHARED((_S_PAD, _CW), jnp.float32),
            pltpu.SemaphoreType.DMA,
            pltpu.SemaphoreType.DMA,
            pltpu.SemaphoreType.DMA,
            pltpu.SemaphoreType.DMA,
            pltpu.SemaphoreType.DMA,
        ],
    )(tbl_l, tbl_r, batch_node_ids, seg_ids3, zrow, zcnt, ones)

    out = pl.pallas_call(
        _finalize_body,
        out_shape=jax.ShapeDtypeStruct((_S, _D), jnp.float32),
    )(psums, pcnts)
    return out


def kernel(node_feature, batch_node_ids, batch_macro_node_ids):
    return _impl(node_feature, batch_node_ids, batch_macro_node_ids)


# speedup vs baseline: 12.8501x; 1.0161x over previous
import jax
import jax.numpy as jnp
from jax import lax
from jax.experimental import pallas as pl
from jax.experimental.pallas import tpu as pltpu
from jax.experimental.pallas import tpu_sc as plsc

_N_NODES = 10000
_D = 128
_DH = _D // 2
_M = 320000
_S = 5000
_NC, _NS = 2, 16
_S_PAD = 5120
_ROWS_PER_TILE = _S_PAD // _NS
_PER_T = _M // _NS
_C = 80
_NCHUNK = _PER_T // _C
_K = 5
_NGRP = _NCHUNK // _K
_L = 16


def _sc_body(tbl_l, tbl_r, node_ids, seg_ids3, zrow,
             psums, pcnts, *scratch):
    idx_n, idx_s, cnt_loc = scratch[:3]
    rows = scratch[3:3 + 2 * _K]
    sums_sp = scratch[3 + 2 * _K]
    gsem = scratch[4 + 2 * _K:6 + 2 * _K]
    ssem = scratch[6 + 2 * _K:8 + 2 * _K]

    cid = lax.axis_index("c")
    sid = lax.axis_index("s")
    base = sid * _PER_T
    r0 = sid * _ROWS_PER_TILE

    pltpu.sync_copy(node_ids.at[pl.ds(base, _PER_T)], idx_n)
    pltpu.sync_copy(seg_ids3.at[sid], idx_s)

    zvec = jnp.zeros((_L,), jnp.float32)

    def zero_cnt(k, carry):
        cnt_loc[pl.ds(k * _L, _L)] = zvec
        return carry

    lax.fori_loop(0, _S_PAD // _L, zero_cnt, 0)
    pltpu.sync_copy(zrow, rows[0])
    for j in range(_ROWS_PER_TILE // _C):
        pltpu.sync_copy(rows[0], sums_sp.at[pl.ds(r0 + j * _C, _C)])
    plsc.subcore_barrier()

    ones_vec = jnp.ones((_L,), jnp.float32)

    def issue_gather(i, buf, sem):
        @pl.when(cid == 0)
        def _():
            pltpu.async_copy(tbl_l.at[idx_n.at[pl.ds(i * _C, _C)]], buf, sem)

        @pl.when(cid == 1)
        def _():
            pltpu.async_copy(tbl_r.at[idx_n.at[pl.ds(i * _C, _C)]], buf, sem)

    for j in range(_K):
        issue_gather(j, rows[j], gsem[0])

    def super_body(u, carry):
        for p in (0, 1):
            t = 2 * u + p
            bb = p * _K
            nbb = (1 - p) * _K
            for j in range(_K):
                pltpu.make_async_copy(zrow, rows[bb + j], gsem[p]).wait()
            for j in range(_K):
                i = t * _K + j
                pltpu.async_copy(rows[bb + j], sums_sp.at[idx_s.at[i]],
                                 ssem[p], add=True)
                for m in range(_C // _L):
                    v = idx_s[i, pl.ds(m * _L, _L)]
                    plsc.addupdate_scatter(cnt_loc, [v], ones_vec)
            def drain_prev():
                for j in range(_K):
                    pltpu.make_async_copy(zrow, rows[nbb + j],
                                          ssem[1 - p]).wait()

            def issue_next():
                for j in range(_K):
                    issue_gather((t + 1) * _K + j, rows[nbb + j],
                                 gsem[1 - p])

            if p == 1:
                drain_prev()
                pl.when(u < (_NGRP // 2) - 1)(issue_next)
            else:
                pl.when(u >= 1)(drain_prev)
                issue_next()
        return carry

    lax.fori_loop(0, _NGRP // 2, super_body, 0)

    for j in range(_K):
        pltpu.make_async_copy(zrow, rows[_K + j], ssem[1]).wait()
    plsc.subcore_barrier()

    pltpu.sync_copy(cnt_loc, pcnts.at[cid, sid])
    for j in range(_ROWS_PER_TILE // _C):
        pltpu.sync_copy(sums_sp.at[pl.ds(r0 + j * _C, _C)], rows[j])
        pltpu.sync_copy(rows[j], psums.at[cid, pl.ds(r0 + j * _C, _C)])


def _finalize_body(ps_ref, pc_ref, out_ref):
    c = 0.5 * jnp.sum(pc_ref[...], axis=(0, 1))[:_S, None]
    inv = 1.0 / jnp.maximum(c, 1.0)
    left = ps_ref[0, :_S, :] * inv
    right = ps_ref[1, :_S, :] * inv
    out_ref[...] = jnp.concatenate([left, right], axis=1)


@jax.jit
def _impl(node_feature, batch_node_ids, batch_macro_node_ids):
    tbl_l = node_feature[:, :_DH]
    tbl_r = node_feature[:, _DH:]
    seg_ids3 = batch_macro_node_ids.reshape(_NS, _NCHUNK, _C)
    zrow = jnp.zeros((_C, _DH), jnp.float32)

    mesh = plsc.VectorSubcoreMesh(core_axis_name="c", subcore_axis_name="s")
    psums, pcnts = pl.kernel(
        _sc_body,
        out_type=(
            jax.ShapeDtypeStruct((_NC, _S_PAD, _DH), jnp.float32),
            jax.ShapeDtypeStruct((_NC, _NS, _S_PAD), jnp.float32),
        ),
        mesh=mesh,
        compiler_params=pltpu.CompilerParams(
            use_tc_tiling_on_sc=False, needs_layout_passes=False),
        scratch_types=[
            pltpu.VMEM((_PER_T,), jnp.int32),
            pltpu.VMEM((_NCHUNK, _C), jnp.int32),
            pltpu.VMEM((_S_PAD,), jnp.float32),
        ] + [pltpu.VMEM((_C, _DH), jnp.float32) for _ in range(2 * _K)] + [
            pltpu.VMEM_SHARED((_S_PAD, _DH), jnp.float32),
            pltpu.SemaphoreType.DMA,
            pltpu.SemaphoreType.DMA,
            pltpu.SemaphoreType.DMA,
            pltpu.SemaphoreType.DMA,
        ],
    )(tbl_l, tbl_r, batch_node_ids, seg_ids3, zrow)

    out = pl.pallas_call(
        _finalize_body,
        out_shape=jax.ShapeDtypeStruct((_S, _D), jnp.float32),
    )(psums, pcnts)
    return out


def kernel(node_feature, batch_node_ids, batch_macro_node_ids):
    return _impl(node_feature, batch_node_ids, batch_macro_node_ids)


# speedup vs baseline: 13.0028x; 1.0119x over previous
import jax
import jax.numpy as jnp
from jax import lax
from jax.experimental import pallas as pl
from jax.experimental.pallas import tpu as pltpu
from jax.experimental.pallas import tpu_sc as plsc

_N_NODES = 10000
_D = 128
_DH = _D // 2
_M = 320000
_S = 5000
_NC, _NS = 2, 16
_S_PAD = 5120
_ROWS_PER_TILE = _S_PAD // _NS
_PER_T = _M // _NS
_C = 80
_NCHUNK = _PER_T // _C
_K = 5
_NGRP = _NCHUNK // _K
_L = 16


def _sc_body(tbl_l, tbl_r, node_ids, seg_ids3, zrow,
             out, *scratch):
    idx_n, idx_s, cnt_loc, cvm, ivm = scratch[:5]
    rows = scratch[5:5 + 2 * _K]
    sums_sp, counts_sp = scratch[5 + 2 * _K:7 + 2 * _K]
    gsem = scratch[7 + 2 * _K:9 + 2 * _K]
    ssem = scratch[9 + 2 * _K:11 + 2 * _K]

    cid = lax.axis_index("c")
    sid = lax.axis_index("s")
    base = sid * _PER_T
    r0 = sid * _ROWS_PER_TILE

    pltpu.sync_copy(node_ids.at[pl.ds(base, _PER_T)], idx_n)
    pltpu.sync_copy(seg_ids3.at[sid], idx_s)

    zvec = jnp.zeros((_L,), jnp.float32)

    def zero_cnt(k, carry):
        cnt_loc[pl.ds(k * _L, _L)] = zvec
        return carry

    lax.fori_loop(0, _S_PAD // _L, zero_cnt, 0)
    pltpu.sync_copy(zrow, rows[0])
    for j in range(_ROWS_PER_TILE // _C):
        pltpu.sync_copy(rows[0], sums_sp.at[pl.ds(r0 + j * _C, _C)])
    plsc.subcore_barrier()

    ones_vec = jnp.ones((_L,), jnp.float32)

    def issue_gather(i, buf, sem):
        @pl.when(cid == 0)
        def _():
            pltpu.async_copy(tbl_l.at[idx_n.at[pl.ds(i * _C, _C)]], buf, sem)

        @pl.when(cid == 1)
        def _():
            pltpu.async_copy(tbl_r.at[idx_n.at[pl.ds(i * _C, _C)]], buf, sem)

    for j in range(_K):
        issue_gather(j, rows[j], gsem[0])

    def super_body(u, carry):
        for p in (0, 1):
            t = 2 * u + p
            bb = p * _K
            nbb = (1 - p) * _K
            for j in range(_K):
                pltpu.make_async_copy(zrow, rows[bb + j], gsem[p]).wait()
            for j in range(_K):
                i = t * _K + j
                pltpu.async_copy(rows[bb + j], sums_sp.at[idx_s.at[i]],
                                 ssem[p], add=True)
                for m in range(_C // _L):
                    v = idx_s[i, pl.ds(m * _L, _L)]
                    plsc.addupdate_scatter(cnt_loc, [v], ones_vec)
            def drain_prev():
                for j in range(_K):
                    pltpu.make_async_copy(zrow, rows[nbb + j],
                                          ssem[1 - p]).wait()

            def issue_next():
                for j in range(_K):
                    issue_gather((t + 1) * _K + j, rows[nbb + j],
                                 gsem[1 - p])

            if p == 1:
                drain_prev()
                pl.when(u < (_NGRP // 2) - 1)(issue_next)
            else:
                pl.when(u >= 1)(drain_prev)
                issue_next()
        return carry

    lax.fori_loop(0, _NGRP // 2, super_body, 0)

    for j in range(_K):
        pltpu.make_async_copy(zrow, rows[_K + j], ssem[1]).wait()

    pltpu.sync_copy(cnt_loc, counts_sp.at[sid])
    plsc.subcore_barrier()

    for r in range(_NS):
        pltpu.sync_copy(counts_sp.at[r, pl.ds(r0, _ROWS_PER_TILE)],
                        cvm.at[r])
    for g in range(_ROWS_PER_TILE // _L):
        acc = cvm[0, pl.ds(g * _L, _L)]
        for r in range(1, _NS):
            acc = acc + cvm[r, pl.ds(g * _L, _L)]
        ivm[pl.ds(g * _L, _L)] = 1.0 / jnp.maximum(acc, 1.0)

    for j in range(_ROWS_PER_TILE // _C):
        pltpu.sync_copy(sums_sp.at[pl.ds(r0 + j * _C, _C)], rows[j])

        def scale_row(r, carry):
            inv = plsc.load_gather(
                ivm, [jnp.full((_L,), j * _C, jnp.int32) + r])
            for m in range(_DH // _L):
                rows[j][r, pl.ds(m * _L, _L)] = (
                    rows[j][r, pl.ds(m * _L, _L)] * inv)
            return carry

        lax.fori_loop(0, _C, scale_row, 0)
        pltpu.sync_copy(
            rows[j],
            out.at[pl.ds(r0 + j * _C, _C), pl.ds(cid * _DH, _DH)])


@jax.jit
def _impl(node_feature, batch_node_ids, batch_macro_node_ids):
    tbl_l = node_feature[:, :_DH]
    tbl_r = node_feature[:, _DH:]
    seg_ids3 = batch_macro_node_ids.reshape(_NS, _NCHUNK, _C)
    zrow = jnp.zeros((_C, _DH), jnp.float32)

    mesh = plsc.VectorSubcoreMesh(core_axis_name="c", subcore_axis_name="s")
    out_pad = pl.kernel(
        _sc_body,
        out_type=jax.ShapeDtypeStruct((_S_PAD, _D), jnp.float32),
        mesh=mesh,
        compiler_params=pltpu.CompilerParams(
            use_tc_tiling_on_sc=False, needs_layout_passes=False),
        scratch_types=[
            pltpu.VMEM((_PER_T,), jnp.int32),
            pltpu.VMEM((_NCHUNK, _C), jnp.int32),
            pltpu.VMEM((_S_PAD,), jnp.float32),
            pltpu.VMEM((_NS, _ROWS_PER_TILE), jnp.float32),
            pltpu.VMEM((_ROWS_PER_TILE,), jnp.float32),
        ] + [pltpu.VMEM((_C, _DH), jnp.float32) for _ in range(2 * _K)] + [
            pltpu.VMEM_SHARED((_S_PAD, _DH), jnp.float32),
            pltpu.VMEM_SHARED((_NS, _S_PAD), jnp.float32),
            pltpu.SemaphoreType.DMA,
            pltpu.SemaphoreType.DMA,
            pltpu.SemaphoreType.DMA,
            pltpu.SemaphoreType.DMA,
        ],
    )(tbl_l, tbl_r, batch_node_ids, seg_ids3, zrow)

    return out_pad[:_S]


def kernel(node_feature, batch_node_ids, batch_macro_node_ids):
    return _impl(node_feature, batch_node_ids, batch_macro_node_ids)


# speedup vs baseline: 13.4418x; 1.0338x over previous
import jax
import jax.numpy as jnp
from jax import lax
from jax.experimental import pallas as pl
from jax.experimental.pallas import tpu as pltpu
from jax.experimental.pallas import tpu_sc as plsc

_N_NODES = 10000
_D = 128
_DH = _D // 2
_M = 320000
_S = 5000
_NC, _NS = 2, 16
_S_PAD = 5120
_ROWS_PER_TILE = _S_PAD // _NS
_PER_T = _M // _NS
_C = 80
_NCHUNK = _PER_T // _C
_K = 5
_NGRP = _NCHUNK // _K
_L = 16


def _sc_body(tbl_l, tbl_r, node_ids, seg_ids3, zrow,
             out, *scratch):
    idx_n, idx_s, cnt_loc, cvm, ivm = scratch[:5]
    rows = scratch[5:5 + 2 * _K]
    sums_sp, counts_sp = scratch[5 + 2 * _K:7 + 2 * _K]
    gsem = scratch[7 + 2 * _K:9 + 2 * _K]
    ssem = scratch[9 + 2 * _K:11 + 2 * _K]

    cid = lax.axis_index("c")
    sid = lax.axis_index("s")
    base = sid * _PER_T
    r0 = sid * _ROWS_PER_TILE

    a = pltpu.async_copy(node_ids.at[pl.ds(base, _PER_T)], idx_n, gsem[0])
    b = pltpu.async_copy(seg_ids3.at[sid], idx_s, gsem[1])
    d = pltpu.async_copy(zrow, rows[0], ssem[1])
    zvec = jnp.zeros((_L,), jnp.float32)

    def zero_cnt(k, carry):
        cnt_loc[pl.ds(k * _L, _L)] = zvec
        return carry

    lax.fori_loop(0, _S_PAD // _L, zero_cnt, 0)
    a.wait()
    b.wait()
    d.wait()
    for j in range(_ROWS_PER_TILE // _C):
        pltpu.sync_copy(rows[0], sums_sp.at[pl.ds(r0 + j * _C, _C)])
    plsc.subcore_barrier()

    ones_vec = jnp.ones((_L,), jnp.float32)

    def issue_gather(i, buf, sem):
        @pl.when(cid == 0)
        def _():
            pltpu.async_copy(tbl_l.at[idx_n.at[pl.ds(i * _C, _C)]], buf, sem)

        @pl.when(cid == 1)
        def _():
            pltpu.async_copy(tbl_r.at[idx_n.at[pl.ds(i * _C, _C)]], buf, sem)

    for j in range(_K):
        issue_gather(j, rows[j], gsem[0])

    def super_body(u, carry):
        for p in (0, 1):
            t = 2 * u + p
            bb = p * _K
            nbb = (1 - p) * _K
            for j in range(_K):
                pltpu.make_async_copy(zrow, rows[bb + j], gsem[p]).wait()
            for j in range(_K):
                i = t * _K + j
                pltpu.async_copy(rows[bb + j], sums_sp.at[idx_s.at[i]],
                                 ssem[p], add=True)
                for m in range(_C // _L):
                    v = idx_s[i, pl.ds(m * _L, _L)]
                    plsc.addupdate_scatter(cnt_loc, [v], ones_vec)
            def drain_prev():
                for j in range(_K):
                    pltpu.make_async_copy(zrow, rows[nbb + j],
                                          ssem[1 - p]).wait()

            def issue_next():
                for j in range(_K):
                    issue_gather((t + 1) * _K + j, rows[nbb + j],
                                 gsem[1 - p])

            if p == 1:
                drain_prev()
                pl.when(u < (_NGRP // 2) - 1)(issue_next)
            else:
                pl.when(u >= 1)(drain_prev)
                issue_next()
        return carry

    lax.fori_loop(0, _NGRP // 2, super_body, 0)

    hist_pub = pltpu.async_copy(cnt_loc, counts_sp.at[sid], gsem[0])
    for j in range(_K):
        pltpu.make_async_copy(zrow, rows[_K + j], ssem[1]).wait()
    hist_pub.wait()
    plsc.subcore_barrier()

    cv = pltpu.async_copy(counts_sp.at[:, pl.ds(r0, _ROWS_PER_TILE)], cvm,
                          gsem[1])
    for j in range(_ROWS_PER_TILE // _C):
        pltpu.async_copy(sums_sp.at[pl.ds(r0 + j * _C, _C)], rows[j],
                         gsem[0])
    cv.wait()
    for g in range(_ROWS_PER_TILE // _L):
        acc = cvm[0, pl.ds(g * _L, _L)]
        for r in range(1, _NS):
            acc = acc + cvm[r, pl.ds(g * _L, _L)]
        ivm[pl.ds(g * _L, _L)] = 1.0 / jnp.maximum(acc, 1.0)

    for j in range(_ROWS_PER_TILE // _C):
        pltpu.make_async_copy(sums_sp.at[pl.ds(r0 + j * _C, _C)], rows[j],
                              gsem[0]).wait()

        def scale_row(r, carry):
            inv = plsc.load_gather(
                ivm, [jnp.full((_L,), j * _C, jnp.int32) + r])
            for m in range(_DH // _L):
                rows[j][r, pl.ds(m * _L, _L)] = (
                    rows[j][r, pl.ds(m * _L, _L)] * inv)
            return carry

        lax.fori_loop(0, _C, scale_row, 0)
        pltpu.async_copy(
            rows[j],
            out.at[pl.ds(r0 + j * _C, _C), pl.ds(cid * _DH, _DH)],
            ssem[0])
    for j in range(_ROWS_PER_TILE // _C):
        pltpu.make_async_copy(
            rows[j],
            out.at[pl.ds(r0 + j * _C, _C), pl.ds(cid * _DH, _DH)],
            ssem[0]).wait()


@jax.jit
def _impl(node_feature, batch_node_ids, batch_macro_node_ids):
    tbl_l = node_feature[:, :_DH]
    tbl_r = node_feature[:, _DH:]
    seg_ids3 = batch_macro_node_ids.reshape(_NS, _NCHUNK, _C)
    zrow = jnp.zeros((_C, _DH), jnp.float32)

    mesh = plsc.VectorSubcoreMesh(core_axis_name="c", subcore_axis_name="s")
    out_pad = pl.kernel(
        _sc_body,
        out_type=jax.ShapeDtypeStruct((_S_PAD, _D), jnp.float32),
        mesh=mesh,
        compiler_params=pltpu.CompilerParams(
            use_tc_tiling_on_sc=False, needs_layout_passes=False),
        scratch_types=[
            pltpu.VMEM((_PER_T,), jnp.int32),
            pltpu.VMEM((_NCHUNK, _C), jnp.int32),
            pltpu.VMEM((_S_PAD,), jnp.float32),
            pltpu.VMEM((_NS, _ROWS_PER_TILE), jnp.float32),
            pltpu.VMEM((_ROWS_PER_TILE,), jnp.float32),
        ] + [pltpu.VMEM((_C, _DH), jnp.float32) for _ in range(2 * _K)] + [
            pltpu.VMEM_SHARED((_S_PAD, _DH), jnp.float32),
            pltpu.VMEM_SHARED((_NS, _S_PAD), jnp.float32),
            pltpu.SemaphoreType.DMA,
            pltpu.SemaphoreType.DMA,
            pltpu.SemaphoreType.DMA,
            pltpu.SemaphoreType.DMA,
        ],
    )(tbl_l, tbl_r, batch_node_ids, seg_ids3, zrow)

    return out_pad[:_S]


def kernel(node_feature, batch_node_ids, batch_macro_node_ids):
    return _impl(node_feature, batch_node_ids, batch_macro_node_ids)
